# baseline (speedup 1.0000x reference)
import jax
import jax.numpy as jnp
from jax import lax
from jax.experimental import pallas as pl
from jax.experimental.pallas import tpu as pltpu
from jax.experimental.pallas import tpu_sc as plsc

N_NODES = 10000
N_EDGES = 320000
B = 8
BFP = 24
WIN = 80
NROWS = N_EDGES // WIN
NC, NS = 2, 16
HDEG_ROWS = NROWS // NS // 2
MSG_ROWS = NROWS // (NC * NS)
NODE_CHUNK = N_NODES // NS
ALN_CHUNK = 1000
SRC_SHIFT = 14

COL_OFFS = sorted(set(list(range(0, BFP - 15, 16)) + [BFP - 16]))


def _sc_body(x2_hbm, ep2_hbm, y_out, w_out, dinv_out,
             src_v, dst_v, dinv_v, norm_v, ones_v,
             rows_a, rows_b, msg_a, msg_b, zy_v, z1_v, xloc_v,
             hzsem, yzsem, xsem, esem, dsem, gsem_a, gsem_b,
             ysem_a, ysem_b, wsem,
             hist_sh, w_sh, y_sh, x_sh):
    cid = lax.axis_index("c")
    sid = lax.axis_index("s")
    nzt = N_NODES // ALN_CHUNK

    def z1_body(j, _):
        z1_v[pl.ds(j * 16, 16)] = jnp.zeros((16,), jnp.float32)
        return 0
    lax.fori_loop(0, 64, z1_body, 0)

    def ones_body(j, _):
        ones_v[pl.ds(j * 16, 16)] = jnp.ones((16,), jnp.float32)
        return 0
    lax.fori_loop(0, WIN // 16, ones_body, 0)

    nslice = pl.ds(sid * NODE_CHUNK, NODE_CHUNK)
    aslice = pl.ds(sid * ALN_CHUNK, ALN_CHUNK)

    def unpack_half(nwin):
        def unpack_body(j, _):
            v = src_v[j // 5, pl.ds((j % 5) * 16, 16)]
            dst_v[j // 5, pl.ds((j % 5) * 16, 16)] = v & jnp.int32(
                (1 << SRC_SHIFT) - 1)
            src_v[j // 5, pl.ds((j % 5) * 16, 16)] = v >> SRC_SHIFT
            return 0
        lax.fori_loop(0, nwin * (WIN // 16), unpack_body, 0)

    for br in range(2):
        x_hbm = x2_hbm.at[br]
        ep_hbm = ep2_hbm.at[br]

        def zy_body(j, _):
            for o in COL_OFFS:
                zy_v[j, pl.ds(o, 16)] = jnp.zeros((16,), jnp.float32)
            return 0
        lax.fori_loop(0, NODE_CHUNK, zy_body, 0)

        pltpu.async_copy(zy_v, y_sh.at[nslice], yzsem)
        pltpu.async_copy(x_hbm.at[nslice], x_sh.at[nslice], xsem)

        @pl.when(sid < nzt)
        def _():
            pltpu.async_copy(z1_v.at[pl.ds(0, ALN_CHUNK)], hist_sh.at[aslice],
                             hzsem)
            pltpu.async_copy(z1_v.at[pl.ds(0, ALN_CHUNK)], w_sh.at[aslice],
                             yzsem)
            pltpu.make_async_copy(z1_v.at[pl.ds(0, ALN_CHUNK)],
                                  hist_sh.at[aslice], hzsem).wait()
        plsc.subcore_barrier()

        DEG_LAG = 8
        for half in range(2):
            hslice = pl.ds(sid * (2 * HDEG_ROWS) + half * HDEG_ROWS,
                           HDEG_ROWS)
            pltpu.sync_copy(ep_hbm.at[hslice], src_v)
            unpack_half(HDEG_ROWS)

            def deg_body(w, _):
                pltpu.async_copy(ones_v, hist_sh.at[dst_v.at[w]], dsem,
                                 add=True)

                @pl.when(w >= DEG_LAG)
                def _():
                    pltpu.make_async_copy(ones_v, hist_sh.at[dst_v.at[0]],
                                          dsem).wait()
                return 0
            lax.fori_loop(0, HDEG_ROWS, deg_body, 0)

            def deg_drain(w, _):
                pltpu.make_async_copy(ones_v, hist_sh.at[dst_v.at[0]],
                                      dsem).wait()
                return 0
            lax.fori_loop(0, DEG_LAG, deg_drain, 0)

        pltpu.make_async_copy(zy_v, y_sh.at[nslice], yzsem).wait()
        pltpu.make_async_copy(x_hbm.at[nslice], x_sh.at[nslice], xsem).wait()

        @pl.when(sid < nzt)
        def _():
            pltpu.make_async_copy(z1_v.at[pl.ds(0, ALN_CHUNK)],
                                  w_sh.at[aslice], yzsem).wait()
        plsc.subcore_barrier()

        pltpu.sync_copy(hist_sh, dinv_v)

        def dinv_body(j, _):
            d = dinv_v[pl.ds(j * 16, 16)] + 1.0
            ii = plsc.bitcast(d, jnp.int32)
            ii = jnp.int32(0x5F3759DF) - (ii >> 1)
            y = plsc.bitcast(ii, jnp.float32)
            for _ in range(3):
                y = y * (1.5 - 0.5 * d * y * y)
            dinv_v[pl.ds(j * 16, 16)] = y
            return 0
        lax.fori_loop(0, N_NODES // 16, dinv_body, 0)

        ebase = cid * (MSG_ROWS * NS) + sid * MSG_ROWS
        pltpu.sync_copy(ep_hbm.at[pl.ds(ebase, MSG_ROWS)], src_v)
        unpack_half(MSG_ROWS)

        def norm_body(w, _):
            def jb(j, _):
                sv = src_v[w, pl.ds(j * 16, 16)]
                dv = dst_v[w, pl.ds(j * 16, 16)]
                gs = plsc.load_gather(dinv_v, [sv])
                gd = plsc.load_gather(dinv_v, [dv])
                norm_v[w, pl.ds(j * 16, 16)] = gs * gd
                return 0
            lax.fori_loop(0, WIN // 16, jb, 0)
            return 0
        lax.fori_loop(0, MSG_ROWS, norm_body, 0)

        bufs = ((rows_a, msg_a, gsem_a, ysem_a),
                (rows_b, msg_b, gsem_b, ysem_b))

        def scale_window(w, rows, msg):
            def mul_body(j, _):
                nv = norm_v[w, pl.ds(j * 16, 16)]
                for l in range(16):
                    s = nv[l]
                    row = j * 16 + l
                    for o in COL_OFFS:
                        msg[row, pl.ds(o, 16)] = rows[row, pl.ds(o, 16)] * s
                return 0
            lax.fori_loop(0, WIN // 16, mul_body, 0)

        for b, (rows, msg, gsem, ysem) in enumerate(bufs):
            pltpu.async_copy(x_sh.at[src_v.at[b]], rows, gsem)

        def pair_body(i, _):
            for b, (rows, msg, gsem, ysem) in enumerate(bufs):
                w = 2 * i + b
                pltpu.make_async_copy(x_sh.at[src_v.at[w]], rows,
                                      gsem).wait()

                @pl.when(i > 0)
                def _():
                    pltpu.make_async_copy(msg, y_sh.at[dst_v.at[w]],
                                          ysem).wait()

                scale_window(w, rows, msg)

                @pl.when(w + 2 < MSG_ROWS)
                def _():
                    pltpu.async_copy(x_sh.at[src_v.at[w + 2]], rows, gsem)

                pltpu.async_copy(msg, y_sh.at[dst_v.at[w]], ysem, add=True)
                pltpu.async_copy(norm_v.at[w], w_sh.at[src_v.at[w]], wsem,
                                 add=True)
            return 0
        lax.fori_loop(0, MSG_ROWS // 2, pair_body, 0)

        if MSG_ROWS % 2 == 1:
            w = MSG_ROWS - 1
            rows, msg, gsem, ysem = bufs[w % 2]
            pltpu.make_async_copy(x_sh.at[src_v.at[w]], rows, gsem).wait()
            pltpu.make_async_copy(msg, y_sh.at[dst_v.at[w]], ysem).wait()
            scale_window(w, rows, msg)
            pltpu.async_copy(msg, y_sh.at[dst_v.at[w]], ysem, add=True)
            pltpu.async_copy(norm_v.at[w], w_sh.at[src_v.at[w]], wsem,
                             add=True)

        for b, (rows, msg, gsem, ysem) in enumerate(bufs):
            pltpu.make_async_copy(msg, y_sh.at[dst_v.at[0]], ysem).wait()

        def w_drain(w, _):
            pltpu.make_async_copy(norm_v.at[0], w_sh.at[src_v.at[0]],
                                  wsem).wait()
            return 0
        lax.fori_loop(0, MSG_ROWS, w_drain, 0)

        plsc.subcore_barrier()

        pltpu.sync_copy(y_sh.at[nslice], zy_v)

        @pl.when(cid == 0)
        def _():
            pltpu.sync_copy(x_sh.at[nslice], xloc_v)

            def self_body(j, _):
                dvec = dinv_v[pl.ds(sid * NODE_CHUNK + j * 16, 16)]
                for l in range(16):
                    s = dvec[l] * dvec[l]
                    row = j * 16 + l
                    for o in COL_OFFS:
                        zy_v[row, pl.ds(o, 16)] = (
                            zy_v[row, pl.ds(o, 16)]
                            + xloc_v[row, pl.ds(o, 16)] * s)
                return 0
            lax.fori_loop(0, NODE_CHUNK // 16, self_body, 0)
            dvec = dinv_v[pl.ds(sid * NODE_CHUNK + NODE_CHUNK - 16, 16)]
            s = dvec[15] * dvec[15]
            row = NODE_CHUNK - 1
            for o in COL_OFFS:
                zy_v[row, pl.ds(o, 16)] = (zy_v[row, pl.ds(o, 16)]
                                           + xloc_v[row, pl.ds(o, 16)] * s)

        pltpu.sync_copy(zy_v, y_out.at[br, cid, nslice])

        @pl.when(sid < nzt)
        def _():
            pltpu.sync_copy(w_sh.at[aslice], w_out.at[br, cid, aslice])

        @pl.when((cid == 0) & (sid < nzt))
        def _():
            pltpu.sync_copy(dinv_v.at[aslice], dinv_out.at[br, aslice])

        plsc.subcore_barrier()


_SC_MESH = plsc.VectorSubcoreMesh(core_axis_name="c", subcore_axis_name="s",
                                  num_cores=NC, num_subcores=NS)
_F32 = jnp.float32
_sc_kernel = pl.kernel(
    _sc_body,
    out_type=(jax.ShapeDtypeStruct((2, NC, N_NODES, BFP), _F32),
              jax.ShapeDtypeStruct((2, NC, N_NODES), _F32),
              jax.ShapeDtypeStruct((2, N_NODES), _F32)),
    mesh=_SC_MESH,
    compiler_params=pltpu.CompilerParams(use_tc_tiling_on_sc=False,
                                         needs_layout_passes=False),
    scratch_types=[
        pltpu.VMEM((MSG_ROWS, WIN), jnp.int32),
        pltpu.VMEM((MSG_ROWS, WIN), jnp.int32),
        pltpu.VMEM((N_NODES,), _F32),
        pltpu.VMEM((MSG_ROWS, WIN), _F32),
        pltpu.VMEM((WIN,), _F32),
        pltpu.VMEM((WIN, BFP), _F32),
        pltpu.VMEM((WIN, BFP), _F32),
        pltpu.VMEM((WIN, BFP), _F32),
        pltpu.VMEM((WIN, BFP), _F32),
        pltpu.VMEM((NODE_CHUNK, BFP), _F32),
        pltpu.VMEM((1024,), _F32),
        pltpu.VMEM((NODE_CHUNK, BFP), _F32),
        pltpu.SemaphoreType.DMA,
        pltpu.SemaphoreType.DMA,
        pltpu.SemaphoreType.DMA,
        pltpu.SemaphoreType.DMA,
        pltpu.SemaphoreType.DMA,
        pltpu.SemaphoreType.DMA,
        pltpu.SemaphoreType.DMA,
        pltpu.SemaphoreType.DMA,
        pltpu.SemaphoreType.DMA,
        pltpu.SemaphoreType.DMA,
        pltpu.VMEM_SHARED((N_NODES,), _F32),
        pltpu.VMEM_SHARED((N_NODES,), _F32),
        pltpu.VMEM_SHARED((N_NODES, BFP), _F32),
        pltpu.VMEM_SHARED((N_NODES, BFP), _F32),
    ],
)


def _leaky(x):
    return jnp.where(x > 0, x, 0.1 * x)


def _tc_body(y2, w2, dinv2, drone,
             W1v, b1v, W2v, b2v, W1e, b1e, W2e, b2e,
             Wmi, bmi, Wmh, bmh, Wmo, bmo, ff_out, out_out):
    C = 1000
    HB = B * 150

    def branch(bidx, W1bd_ref, b1t_ref):
        W1bd = W1bd_ref[...]
        b1t = b1t_ref[...]

        def chunk(i, acc):
            sl = pl.ds(i * C, C)
            Y = y2[bidx, 0, sl, :] + y2[bidx, 1, sl, :]
            d = dinv2[sl, bidx:bidx + 1]
            wp = w2[sl, :]
            wt = (wp[:, 2 * bidx:2 * bidx + 1] + wp[:, 2 * bidx + 1:2 * bidx + 2]
                  + d * d) * (1.0 / N_NODES)
            Z = jnp.dot(Y, W1bd, preferred_element_type=jnp.float32) + b1t
            Z = _leaky(Z)
            return acc + jnp.sum(wt * Z, axis=0, keepdims=True)

        S = lax.fori_loop(0, N_NODES // C, chunk,
                          jnp.zeros((1, HB), jnp.float32))
        return jnp.concatenate(
            [S[:, b * 150:(b + 1) * 150] for b in range(B)], axis=0)

    Sv = branch(0, W1v, b1v)
    Se = branch(1, W1e, b1e)
    v_emb = jnp.dot(Sv, W2v[...], preferred_element_type=jnp.float32) + b2v[...]
    e_emb = jnp.dot(Se, W2e[...], preferred_element_type=jnp.float32) + b2e[...]
    ff = jnp.concatenate([v_emb, e_emb, drone[...]], axis=1)
    ff_out[...] = ff
    h = _leaky(jnp.dot(ff, Wmi[...], preferred_element_type=jnp.float32) + bmi[...])
    h = _leaky(jnp.dot(h, Wmh[...], preferred_element_type=jnp.float32) + bmh[...])
    out_out[...] = jnp.tanh(jnp.dot(h, Wmo[...], preferred_element_type=jnp.float32) + bmo[...])


def _pack_x(x):
    n = x.shape[1]
    xp = jnp.transpose(x, (1, 0, 2)).reshape(n, -1)
    if xp.shape[1] < BFP:
        xp = jnp.pad(xp, ((0, 0), (0, BFP - xp.shape[1])))
    return xp


def _block_diag_w1(W1):
    F = W1.shape[0]
    Wbd = jnp.zeros((BFP, B * 150), W1.dtype)
    for b in range(B):
        Wbd = Wbd.at[b * F:(b + 1) * F, b * 150:(b + 1) * 150].set(W1)
    return Wbd


def kernel(vertiport_features, vertiport_edge, evtol_features, evtol_edge,
           next_drone_embedding, Wv1, bv1, Wv2, bv2, We1, be1, We2, be2,
           Wmi, bmi, Wmh, bmh, Wmo, bmo):
    x2 = jnp.stack([_pack_x(vertiport_features), _pack_x(evtol_features)])
    mask = jnp.int32((1 << SRC_SHIFT) - 1)

    def pack_edges(e):
        src = e[0, 0].astype(jnp.int32)
        dst = e[0, 1].astype(jnp.int32)
        return ((src << SRC_SHIFT) | (dst & mask)).reshape(NROWS, WIN)

    ep2 = jnp.stack([pack_edges(vertiport_edge), pack_edges(evtol_edge)])

    y2, w2, dinv2 = _sc_kernel(x2, ep2)

    W1v = _block_diag_w1(Wv1)
    W1e = _block_diag_w1(We1)
    b1v = jnp.tile(bv1, B).reshape(1, B * 150)
    b1e = jnp.tile(be1, B).reshape(1, B * 150)

    ff, out = pl.pallas_call(
        _tc_body,
        out_shape=(jax.ShapeDtypeStruct((B, 134), jnp.float32),
                   jax.ShapeDtypeStruct((B, 4), jnp.float32)),
    )(y2, jnp.transpose(w2, (2, 0, 1)).reshape(N_NODES, 4),
      jnp.transpose(dinv2), next_drone_embedding,
      W1v, b1v, Wv2, bv2.reshape(1, 64),
      W1e, b1e, We2, be2.reshape(1, 64),
      Wmi, bmi.reshape(1, 128), Wmh, bmh.reshape(1, 64),
      Wmo, bmo.reshape(1, 4))
    return (ff, out)


# speedup vs baseline: 1.2168x; 1.2168x over previous
import functools

import jax
import jax.numpy as jnp
from jax import lax
from jax.experimental import pallas as pl
from jax.experimental.pallas import tpu as pltpu
from jax.experimental.pallas import tpu_sc as plsc

N_NODES = 10000
N_EDGES = 320000
B = 8
WIN = 80
NROWS = N_EDGES // WIN
NC, NS = 2, 16
DEG_ROWS = NROWS // NS
MSG_ROWS = NROWS // (NC * NS)
NODE_CHUNK = N_NODES // NS
ALN_CHUNK = 1000


def _sc_body(BFp, x_hbm, src_hbm, dst_hbm, y_out, w_out, dinv_out,
             degidx_v, src_v, dst_v, dinv_v, norm_v, ones_v,
             rows_a, rows_b, msg_a, msg_b, zy_v, z1_v,
             hzsem, yzsem, xsem, esem, dsem, gsem_a, gsem_b,
             ysem_a, ysem_b, wsem,
             hist_sh, w_sh, y_sh, x_sh):
    cid = lax.axis_index("c")
    sid = lax.axis_index("s")
    nzt = N_NODES // ALN_CHUNK

    offs = sorted(set(list(range(0, BFp - 15, 16)) + [BFp - 16]))

    def zy_body(j, _):
        for o in offs:
            zy_v[j, pl.ds(o, 16)] = jnp.zeros((16,), jnp.float32)
        return 0
    lax.fori_loop(0, NODE_CHUNK, zy_body, 0)

    def z1_body(j, _):
        z1_v[pl.ds(j * 16, 16)] = jnp.zeros((16,), jnp.float32)
        return 0
    lax.fori_loop(0, 64, z1_body, 0)

    def ones_body(j, _):
        ones_v[pl.ds(j * 16, 16)] = jnp.ones((16,), jnp.float32)
        return 0
    lax.fori_loop(0, WIN // 16, ones_body, 0)

    nslice = pl.ds(sid * NODE_CHUNK, NODE_CHUNK)
    aslice = pl.ds(sid * ALN_CHUNK, ALN_CHUNK)
    pltpu.async_copy(zy_v, y_sh.at[nslice], yzsem)
    pltpu.async_copy(x_hbm.at[nslice], x_sh.at[nslice], xsem)

    @pl.when(sid < nzt)
    def _():
        pltpu.async_copy(z1_v.at[pl.ds(0, ALN_CHUNK)], hist_sh.at[aslice],
                         hzsem)
        pltpu.async_copy(z1_v.at[pl.ds(0, ALN_CHUNK)], w_sh.at[aslice],
                         yzsem)

    pltpu.async_copy(dst_hbm.at[pl.ds(sid * DEG_ROWS, DEG_ROWS)], degidx_v,
                     esem)
    ebase = cid * (MSG_ROWS * NS) + sid * MSG_ROWS
    pltpu.async_copy(src_hbm.at[pl.ds(ebase, MSG_ROWS)], src_v, esem)
    pltpu.async_copy(dst_hbm.at[pl.ds(ebase, MSG_ROWS)], dst_v, esem)

    @pl.when(sid < nzt)
    def _():
        pltpu.make_async_copy(z1_v.at[pl.ds(0, ALN_CHUNK)],
                              hist_sh.at[aslice], hzsem).wait()
    plsc.subcore_barrier()

    pltpu.make_async_copy(dst_hbm.at[pl.ds(sid * DEG_ROWS, DEG_ROWS)],
                          degidx_v, esem).wait()
    pltpu.make_async_copy(src_hbm.at[pl.ds(ebase, MSG_ROWS)], src_v,
                          esem).wait()
    pltpu.make_async_copy(dst_hbm.at[pl.ds(ebase, MSG_ROWS)], dst_v,
                          esem).wait()

    DEG_LAG = 8

    def deg_body(w, _):
        pltpu.async_copy(ones_v, hist_sh.at[degidx_v.at[w]], dsem, add=True)

        @pl.when(w >= DEG_LAG)
        def _():
            pltpu.make_async_copy(ones_v, hist_sh.at[degidx_v.at[0]],
                                  dsem).wait()
        return 0
    lax.fori_loop(0, DEG_ROWS, deg_body, 0)

    def deg_drain(w, _):
        pltpu.make_async_copy(ones_v, hist_sh.at[degidx_v.at[0]], dsem).wait()
        return 0
    lax.fori_loop(0, DEG_LAG, deg_drain, 0)

    pltpu.make_async_copy(zy_v, y_sh.at[nslice], yzsem).wait()
    pltpu.make_async_copy(x_hbm.at[nslice], x_sh.at[nslice], xsem).wait()

    @pl.when(sid < nzt)
    def _():
        pltpu.make_async_copy(z1_v.at[pl.ds(0, ALN_CHUNK)], w_sh.at[aslice],
                              yzsem).wait()
    plsc.subcore_barrier()

    pltpu.sync_copy(hist_sh, dinv_v)

    def dinv_body(j, _):
        d = dinv_v[pl.ds(j * 16, 16)] + 1.0
        ii = plsc.bitcast(d, jnp.int32)
        ii = jnp.int32(0x5F3759DF) - (ii >> 1)
        y = plsc.bitcast(ii, jnp.float32)
        for _ in range(3):
            y = y * (1.5 - 0.5 * d * y * y)
        dinv_v[pl.ds(j * 16, 16)] = y
        return 0
    lax.fori_loop(0, N_NODES // 16, dinv_body, 0)

    def norm_body(w, _):
        def jb(j, _):
            sv = src_v[w, pl.ds(j * 16, 16)]
            dv = dst_v[w, pl.ds(j * 16, 16)]
            gs = plsc.load_gather(dinv_v, [sv])
            gd = plsc.load_gather(dinv_v, [dv])
            norm_v[w, pl.ds(j * 16, 16)] = gs * gd
            return 0
        lax.fori_loop(0, WIN // 16, jb, 0)
        return 0
    lax.fori_loop(0, MSG_ROWS, norm_body, 0)

    bufs = ((rows_a, msg_a, gsem_a, ysem_a), (rows_b, msg_b, gsem_b, ysem_b))

    def scale_window(w, rows, msg):
        def mul_body(j, _):
            nv = norm_v[w, pl.ds(j * 16, 16)]
            for l in range(16):
                s = nv[l]
                row = j * 16 + l
                for o in offs:
                    msg[row, pl.ds(o, 16)] = rows[row, pl.ds(o, 16)] * s
            return 0
        lax.fori_loop(0, WIN // 16, mul_body, 0)

    for b, (rows, msg, gsem, ysem) in enumerate(bufs):
        pltpu.async_copy(x_sh.at[src_v.at[b]], rows, gsem)

    def pair_body(i, _):
        for b, (rows, msg, gsem, ysem) in enumerate(bufs):
            w = 2 * i + b
            pltpu.make_async_copy(x_sh.at[src_v.at[w]], rows, gsem).wait()

            @pl.when(i > 0)
            def _():
                pltpu.make_async_copy(msg, y_sh.at[dst_v.at[w]], ysem).wait()

            scale_window(w, rows, msg)

            @pl.when(w + 2 < MSG_ROWS)
            def _():
                pltpu.async_copy(x_sh.at[src_v.at[w + 2]], rows, gsem)

            pltpu.async_copy(msg, y_sh.at[dst_v.at[w]], ysem, add=True)
            pltpu.async_copy(norm_v.at[w], w_sh.at[src_v.at[w]], wsem,
                             add=True)
        return 0
    lax.fori_loop(0, MSG_ROWS // 2, pair_body, 0)

    if MSG_ROWS % 2 == 1:
        w = MSG_ROWS - 1
        rows, msg, gsem, ysem = bufs[w % 2]
        pltpu.make_async_copy(x_sh.at[src_v.at[w]], rows, gsem).wait()
        pltpu.make_async_copy(msg, y_sh.at[dst_v.at[w]], ysem).wait()
        scale_window(w, rows, msg)
        pltpu.async_copy(msg, y_sh.at[dst_v.at[w]], ysem, add=True)
        pltpu.async_copy(norm_v.at[w], w_sh.at[src_v.at[w]], wsem, add=True)

    for b, (rows, msg, gsem, ysem) in enumerate(bufs):
        pltpu.make_async_copy(msg, y_sh.at[dst_v.at[0]], ysem).wait()

    def w_drain(w, _):
        pltpu.make_async_copy(norm_v.at[0], w_sh.at[src_v.at[0]],
                              wsem).wait()
        return 0
    lax.fori_loop(0, MSG_ROWS, w_drain, 0)

    plsc.subcore_barrier()

    pltpu.sync_copy(y_sh.at[pl.ds(sid * NODE_CHUNK, NODE_CHUNK)],
                    y_out.at[cid, pl.ds(sid * NODE_CHUNK, NODE_CHUNK)])

    @pl.when(sid < N_NODES // ALN_CHUNK)
    def _():
        pltpu.sync_copy(w_sh.at[pl.ds(sid * ALN_CHUNK, ALN_CHUNK)],
                        w_out.at[cid, pl.ds(sid * ALN_CHUNK, ALN_CHUNK)])

    @pl.when((cid == 0) & (sid < N_NODES // ALN_CHUNK))
    def _():
        pltpu.sync_copy(dinv_v.at[pl.ds(sid * ALN_CHUNK, ALN_CHUNK)],
                        dinv_out.at[pl.ds(sid * ALN_CHUNK, ALN_CHUNK)])


def _make_sc_kernel(BFp):
    mesh = plsc.VectorSubcoreMesh(core_axis_name="c", subcore_axis_name="s",
                                  num_cores=NC, num_subcores=NS)
    f32 = jnp.float32
    return pl.kernel(
        functools.partial(_sc_body, BFp),
        out_type=(jax.ShapeDtypeStruct((NC, N_NODES, BFp), f32),
                  jax.ShapeDtypeStruct((NC, N_NODES), f32),
                  jax.ShapeDtypeStruct((N_NODES,), f32)),
        mesh=mesh,
        compiler_params=pltpu.CompilerParams(use_tc_tiling_on_sc=False,
                                             needs_layout_passes=False),
        scratch_types=[
            pltpu.VMEM((DEG_ROWS, WIN), jnp.int32),
            pltpu.VMEM((MSG_ROWS, WIN), jnp.int32),
            pltpu.VMEM((MSG_ROWS, WIN), jnp.int32),
            pltpu.VMEM((N_NODES,), f32),
            pltpu.VMEM((MSG_ROWS, WIN), f32),
            pltpu.VMEM((WIN,), f32),
            pltpu.VMEM((WIN, BFp), f32),
            pltpu.VMEM((WIN, BFp), f32),
            pltpu.VMEM((WIN, BFp), f32),
            pltpu.VMEM((WIN, BFp), f32),
            pltpu.VMEM((NODE_CHUNK, BFp), f32),
            pltpu.VMEM((1024,), f32),
            pltpu.SemaphoreType.DMA,
            pltpu.SemaphoreType.DMA,
            pltpu.SemaphoreType.DMA,
            pltpu.SemaphoreType.DMA,
            pltpu.SemaphoreType.DMA,
            pltpu.SemaphoreType.DMA,
            pltpu.SemaphoreType.DMA,
            pltpu.SemaphoreType.DMA,
            pltpu.SemaphoreType.DMA,
            pltpu.SemaphoreType.DMA,
            pltpu.VMEM_SHARED((N_NODES,), f32),
            pltpu.VMEM_SHARED((N_NODES,), f32),
            pltpu.VMEM_SHARED((N_NODES, BFp), f32),
            pltpu.VMEM_SHARED((N_NODES, BFp), f32),
        ],
    )


def _leaky(x):
    return jnp.where(x > 0, x, 0.1 * x)


def _tc_body(yv, wv, dv, xv, ye, we, de, xe, drone,
             W1v, b1v, W2v, b2v, W1e, b1e, W2e, b2e,
             Wmi, bmi, Wmh, bmh, Wmo, bmo, ff_out, out_out):
    C = 1000
    HB = B * 150

    def branch(y_ref, w_ref, dinv_ref, x_ref, W1bd_ref, b1t_ref):
        W1bd = W1bd_ref[...]
        b1t = b1t_ref[...]

        def chunk(i, acc):
            sl = pl.ds(i * C, C)
            d = dinv_ref[sl, :]
            d2 = d * d
            Y = y_ref[0, sl, :] + y_ref[1, sl, :] + d2 * x_ref[sl, :]
            wp = w_ref[sl, :]
            wt = (wp[:, 0:1] + wp[:, 1:2] + d2) * (1.0 / N_NODES)
            Z = jnp.dot(Y, W1bd, preferred_element_type=jnp.float32) + b1t
            Z = _leaky(Z)
            return acc + jnp.sum(wt * Z, axis=0, keepdims=True)

        S = lax.fori_loop(0, N_NODES // C, chunk,
                          jnp.zeros((1, HB), jnp.float32))
        return jnp.concatenate(
            [S[:, b * 150:(b + 1) * 150] for b in range(B)], axis=0)

    Sv = branch(yv, wv, dv, xv, W1v, b1v)
    Se = branch(ye, we, de, xe, W1e, b1e)
    v_emb = jnp.dot(Sv, W2v[...], preferred_element_type=jnp.float32) + b2v[...]
    e_emb = jnp.dot(Se, W2e[...], preferred_element_type=jnp.float32) + b2e[...]
    ff = jnp.concatenate([v_emb, e_emb, drone[...]], axis=1)
    ff_out[...] = ff
    h = _leaky(jnp.dot(ff, Wmi[...], preferred_element_type=jnp.float32) + bmi[...])
    h = _leaky(jnp.dot(h, Wmh[...], preferred_element_type=jnp.float32) + bmh[...])
    out_out[...] = jnp.tanh(jnp.dot(h, Wmo[...], preferred_element_type=jnp.float32) + bmo[...])


def _pack_x(x, BFp):
    n = x.shape[1]
    xp = jnp.transpose(x, (1, 0, 2)).reshape(n, -1)
    if xp.shape[1] < BFp:
        xp = jnp.pad(xp, ((0, 0), (0, BFp - xp.shape[1])))
    return xp


def _block_diag_w1(W1, BFp):
    F = W1.shape[0]
    Wbd = jnp.zeros((BFp, B * 150), W1.dtype)
    for b in range(B):
        Wbd = Wbd.at[b * F:(b + 1) * F, b * 150:(b + 1) * 150].set(W1)
    return Wbd


def kernel(vertiport_features, vertiport_edge, evtol_features, evtol_edge,
           next_drone_embedding, Wv1, bv1, Wv2, bv2, We1, be1, We2, be2,
           Wmi, bmi, Wmh, bmh, Wmo, bmo):
    BFv, BFe = 16, 24
    xv = _pack_x(vertiport_features, BFv)
    xe = _pack_x(evtol_features, BFe)
    srcv = vertiport_edge[0, 0].reshape(NROWS, WIN)
    dstv = vertiport_edge[0, 1].reshape(NROWS, WIN)
    srce = evtol_edge[0, 0].reshape(NROWS, WIN)
    dste = evtol_edge[0, 1].reshape(NROWS, WIN)

    yv, wv, dv = _make_sc_kernel(BFv)(xv, srcv, dstv)
    ye, we, de = _make_sc_kernel(BFe)(xe, srce, dste)

    W1v = _block_diag_w1(Wv1, BFv)
    W1e = _block_diag_w1(We1, BFe)
    b1v = jnp.tile(bv1, B).reshape(1, B * 150)
    b1e = jnp.tile(be1, B).reshape(1, B * 150)

    ff, out = pl.pallas_call(
        _tc_body,
        out_shape=(jax.ShapeDtypeStruct((B, 134), jnp.float32),
                   jax.ShapeDtypeStruct((B, 4), jnp.float32)),
    )(yv, jnp.transpose(wv), dv.reshape(N_NODES, 1), xv,
      ye, jnp.transpose(we), de.reshape(N_NODES, 1), xe,
      next_drone_embedding,
      W1v, b1v, Wv2, bv2.reshape(1, 64),
      W1e, b1e, We2, be2.reshape(1, 64),
      Wmi, bmi.reshape(1, 128), Wmh, bmh.reshape(1, 64),
      Wmo, bmo.reshape(1, 4))
    return (ff, out)
